# Initial kernel scaffold; baseline (speedup 1.0000x reference)
import jax
import jax.numpy as jnp
from jax.experimental import pallas as pl


def kernel(feat_s, feat_t):
    raise NotImplementedError("write your pallas kernel here")


# speedup vs baseline: 35.2336x; 35.2336x over previous
import functools

import jax
import jax.numpy as jnp
from jax import lax
from jax.experimental import pallas as pl
from jax.experimental.pallas import tpu as pltpu
from jax.experimental.pallas import tpu_sc as plsc

_T = 4.0
_NB = 64
_ROWS = 8
_NPR = 96 * 32 * 32
_SPLIT = 4
_CHUNK = _NPR // _SPLIT
_ITERS = _CHUNK // 16
_NC = 2
_NS = 16


def _sc_hist_kernel(fs_hbm, ft_hbm, out_hbm, s_v, t_v, hist_v, res_v, sem):
    wid = lax.axis_index("s") * _NC + lax.axis_index("c")
    row = wid // _SPLIT
    q = wid % _SPLIT
    off = row * _NPR + q * _CHUNK
    cs = pltpu.async_copy(fs_hbm.at[pl.ds(off, _CHUNK)], s_v, sem)
    ct = pltpu.async_copy(ft_hbm.at[pl.ds(off, _CHUNK)], t_v, sem)

    def zero_body(i, carry):
        hist_v[pl.ds(i * 16, 16)] = jnp.zeros((16,), jnp.float32)
        return carry

    lax.fori_loop(0, (2 * 16 * _NB) // 16, zero_body, 0)
    cs.wait()
    ct.wait()

    lane = lax.broadcasted_iota(jnp.int32, (16,), 0)
    base_s = lane * _NB
    base_t = base_s + 16 * _NB
    ones = jnp.ones((16,), jnp.float32)

    def body(i, carry):
        vs = s_v[pl.ds(i * 16, 16)].astype(jnp.int32)
        vt = t_v[pl.ds(i * 16, 16)].astype(jnp.int32)
        plsc.addupdate_scatter(hist_v, [base_s + vs], ones)
        plsc.addupdate_scatter(hist_v, [base_t + vt], ones)
        return carry

    lax.fori_loop(0, _ITERS, body, 0)

    for t in range(2):
        for j in range(4):
            acc = hist_v[pl.ds(t * 16 * _NB + j * 16, 16)]
            for l in range(1, 16):
                acc = acc + hist_v[pl.ds(t * 16 * _NB + l * _NB + j * 16, 16)]
            res_v[pl.ds(t * _NB + j * 16, 16)] = acc

    base = (q * 16 + row) * _NB
    pltpu.sync_copy(res_v.at[pl.ds(0, _NB)], out_hbm.at[pl.ds(base, _NB)])
    pltpu.sync_copy(res_v.at[pl.ds(_NB, _NB)],
                    out_hbm.at[pl.ds(base + 8 * _NB, _NB)])


@functools.cache
def _hist_call():
    return pl.kernel(
        _sc_hist_kernel,
        out_type=jax.ShapeDtypeStruct((4 * 16 * _NB,), jnp.float32),
        mesh=plsc.VectorSubcoreMesh(core_axis_name="c", subcore_axis_name="s"),
        scratch_types=[
            pltpu.VMEM((_CHUNK,), jnp.float32),
            pltpu.VMEM((_CHUNK,), jnp.float32),
            pltpu.VMEM((2 * 16 * _NB,), jnp.float32),
            pltpu.VMEM((2 * _NB,), jnp.float32),
            pltpu.SemaphoreType.DMA,
        ],
        compiler_params=pltpu.CompilerParams(needs_layout_passes=False),
    )


def _compact_logits(h):
    ones8 = jnp.ones((1, _ROWS), jnp.float32)
    tot = jnp.dot(ones8, h, preferred_element_type=jnp.float32)
    pres = (tot > 0.5).astype(jnp.float32)
    iota_w = lax.broadcasted_iota(jnp.int32, (_NB, _NB), 0)
    iota_v = lax.broadcasted_iota(jnp.int32, (_NB, _NB), 1)
    le = (iota_w <= iota_v).astype(jnp.float32)
    cum = jnp.dot(pres, le, preferred_element_type=jnp.float32)
    iota_i = lax.broadcasted_iota(jnp.int32, (_NB, _NB), 0).astype(jnp.float32)
    q = pres * jnp.where(jnp.abs(cum - (iota_i + 1.0)) < 0.5, 1.0, 0.0)
    comp = lax.dot_general(h, q, (((1,), (1,)), ((), ())),
                           preferred_element_type=jnp.float32)
    comp = jnp.floor(comp + 0.5)
    nb = jnp.sum(pres)
    one_bin = nb == 1.0
    logits = jnp.where(one_bin, 0.0, jnp.log(comp + 1e-8))
    iota_row = lax.broadcasted_iota(jnp.int32, (1, _NB), 1).astype(jnp.float32)
    valid = jnp.where(one_bin,
                      jnp.where(iota_row < 2.0, 1.0, 0.0),
                      jnp.where(iota_row < nb, 1.0, 0.0))
    return logits, valid


def _kl_kernel(p_ref, o_ref):
    hist = p_ref[0] + p_ref[1] + p_ref[2] + p_ref[3]
    hs = hist[0:_ROWS]
    ht = hist[_ROWS:2 * _ROWS]
    logit_s, valid_s = _compact_logits(hs)
    logit_t, valid_t = _compact_logits(ht)
    neg = -1e30
    x_s = jnp.where(valid_s > 0.0, logit_s * (1.0 / _T), neg)
    x_t = jnp.where(valid_t > 0.0, logit_t * (1.0 / _T), neg)
    m_s = jnp.max(x_s, axis=1, keepdims=True)
    logp_s = x_s - (m_s + jnp.log(jnp.sum(jnp.exp(x_s - m_s), axis=1,
                                          keepdims=True)))
    m_t = jnp.max(x_t, axis=1, keepdims=True)
    e_t = jnp.exp(x_t - m_t)
    p_t = e_t / jnp.sum(e_t, axis=1, keepdims=True)
    logp_t = jnp.log(p_t)
    logp_t = jnp.where(valid_t > 0.0, logp_t, 0.0)
    valid = valid_s * valid_t
    kl = jnp.where(valid > 0.0, p_t * (logp_t - logp_s), 0.0)
    o_ref[...] = jnp.sum(kl, axis=(0, 1), keepdims=True) * (_T * _T / _ROWS)


def kernel(feat_s, feat_t):
    fs = feat_s.reshape(-1)
    ft = feat_t.reshape(-1)
    partials = _hist_call()(fs, ft).reshape(4, 16, _NB)
    loss = pl.pallas_call(
        _kl_kernel,
        out_shape=jax.ShapeDtypeStruct((1, 1), jnp.float32),
    )(partials)
    return loss[0, 0]


# speedup vs baseline: 54.1572x; 1.5371x over previous
import functools

import jax
import jax.numpy as jnp
from jax import lax
from jax.experimental import pallas as pl
from jax.experimental.pallas import tpu as pltpu
from jax.experimental.pallas import tpu_sc as plsc

_T = 4.0
_NB = 64
_ROWS = 8
_NPR = 96 * 32 * 32
_SPLIT = 4
_CHUNK = _NPR // _SPLIT
_ITERS = _CHUNK // 16
_NC = 2
_NS = 16


def _sc_hist_kernel(fs_hbm, ft_hbm, out_hbm, s_v, t_v, hist_v, res_v, sem):
    wid = lax.axis_index("s") * _NC + lax.axis_index("c")
    row = wid // _SPLIT
    q = wid % _SPLIT
    off = row * _NPR + q * _CHUNK
    cs = pltpu.async_copy(fs_hbm.at[pl.ds(off, _CHUNK)], s_v, sem)
    ct = pltpu.async_copy(ft_hbm.at[pl.ds(off, _CHUNK)], t_v, sem)

    def zero_body(i, carry):
        hist_v[pl.ds(i * 16, 16)] = jnp.zeros((16,), jnp.float32)
        return carry

    lax.fori_loop(0, (2 * 16 * _NB) // 16, zero_body, 0)
    cs.wait()
    ct.wait()

    lane = lax.broadcasted_iota(jnp.int32, (16,), 0)
    base_s = lane * _NB
    base_t = base_s + 16 * _NB
    ones = jnp.ones((16,), jnp.float32)

    def body(i, carry):
        vs = s_v[pl.ds(i * 16, 16)].astype(jnp.int32)
        vt = t_v[pl.ds(i * 16, 16)].astype(jnp.int32)
        plsc.addupdate_scatter(hist_v, [base_s + vs], ones)
        plsc.addupdate_scatter(hist_v, [base_t + vt], ones)
        return carry

    lax.fori_loop(0, _ITERS, body, 0)

    for t in range(2):
        for j in range(4):
            acc = hist_v[pl.ds(t * 16 * _NB + j * 16, 16)]
            for l in range(1, 16):
                acc = acc + hist_v[pl.ds(t * 16 * _NB + l * _NB + j * 16, 16)]
            res_v[pl.ds(t * _NB + j * 16, 16)] = acc

    base = (q * 16 + row) * _NB
    pltpu.sync_copy(res_v.at[pl.ds(0, _NB)], out_hbm.at[pl.ds(base, _NB)])
    pltpu.sync_copy(res_v.at[pl.ds(_NB, _NB)],
                    out_hbm.at[pl.ds(base + 8 * _NB, _NB)])


@functools.cache
def _hist_call():
    return pl.kernel(
        _sc_hist_kernel,
        out_type=jax.ShapeDtypeStruct((4 * 16 * _NB,), jnp.float32),
        mesh=plsc.VectorSubcoreMesh(core_axis_name="c", subcore_axis_name="s"),
        scratch_types=[
            pltpu.VMEM((_CHUNK,), jnp.float32),
            pltpu.VMEM((_CHUNK,), jnp.float32),
            pltpu.VMEM((2 * 16 * _NB,), jnp.float32),
            pltpu.VMEM((2 * _NB,), jnp.float32),
            pltpu.SemaphoreType.DMA,
        ],
        compiler_params=pltpu.CompilerParams(needs_layout_passes=False),
    )


def _compact_logits(h):
    ones8 = jnp.ones((1, _ROWS), jnp.float32)
    tot = jnp.dot(ones8, h, preferred_element_type=jnp.float32)
    pres = (tot > 0.5).astype(jnp.float32)
    iota_w = lax.broadcasted_iota(jnp.int32, (_NB, _NB), 0)
    iota_v = lax.broadcasted_iota(jnp.int32, (_NB, _NB), 1)
    le = (iota_w <= iota_v).astype(jnp.float32)
    cum = jnp.dot(pres, le, preferred_element_type=jnp.float32)
    iota_i = lax.broadcasted_iota(jnp.int32, (_NB, _NB), 0).astype(jnp.float32)
    q = pres * jnp.where(jnp.abs(cum - (iota_i + 1.0)) < 0.5, 1.0, 0.0)
    comp = lax.dot_general(h, q, (((1,), (1,)), ((), ())),
                           preferred_element_type=jnp.float32)
    comp = jnp.floor(comp + 0.5)
    nb = jnp.sum(pres)
    one_bin = nb == 1.0
    logits = jnp.where(one_bin, 0.0, jnp.log(comp + 1e-8))
    iota_row = lax.broadcasted_iota(jnp.int32, (1, _NB), 1).astype(jnp.float32)
    valid = jnp.where(one_bin,
                      jnp.where(iota_row < 2.0, 1.0, 0.0),
                      jnp.where(iota_row < nb, 1.0, 0.0))
    return logits, valid


def _kl_kernel(p_ref, o_ref):
    hist = p_ref[0] + p_ref[1] + p_ref[2] + p_ref[3]
    hs = hist[0:_ROWS]
    ht = hist[_ROWS:2 * _ROWS]
    logit_s, valid_s = _compact_logits(hs)
    logit_t, valid_t = _compact_logits(ht)
    neg = -1e30
    x_s = jnp.where(valid_s > 0.0, logit_s * (1.0 / _T), neg)
    x_t = jnp.where(valid_t > 0.0, logit_t * (1.0 / _T), neg)
    m_s = jnp.max(x_s, axis=1, keepdims=True)
    logp_s = x_s - (m_s + jnp.log(jnp.sum(jnp.exp(x_s - m_s), axis=1,
                                          keepdims=True)))
    m_t = jnp.max(x_t, axis=1, keepdims=True)
    e_t = jnp.exp(x_t - m_t)
    p_t = e_t / jnp.sum(e_t, axis=1, keepdims=True)
    logp_t = jnp.log(p_t)
    logp_t = jnp.where(valid_t > 0.0, logp_t, 0.0)
    valid = valid_s * valid_t
    kl = jnp.where(valid > 0.0, p_t * (logp_t - logp_s), 0.0)
    o_ref[...] = jnp.sum(kl, axis=(0, 1), keepdims=True) * (_T * _T / _ROWS)


def kernel(feat_s, feat_t):
    fs = jnp.transpose(feat_s, (0, 2, 3, 1)).reshape(-1)
    ft = jnp.transpose(feat_t, (0, 2, 3, 1)).reshape(-1)
    partials = _hist_call()(fs, ft).reshape(4, 16, _NB)
    loss = pl.pallas_call(
        _kl_kernel,
        out_shape=jax.ShapeDtypeStruct((1, 1), jnp.float32),
    )(partials)
    return loss[0, 0]
